# baseline (speedup 1.0000x reference)
import functools
import jax
import jax.numpy as jnp
from jax import lax
from jax.experimental import pallas as pl
from jax.experimental.pallas import tpu as pltpu
from jax.experimental.pallas import tpu_sc as plsc

NC, NS, L = 2, 16, 16
NW = NC * NS
D = 64
B, S = 4096, 200
N = B * S
CI = 256
TILES = N // CI
TPW = TILES // NW
SCALE = 8.0

_mesh = plsc.VectorSubcoreMesh(
    core_axis_name="c", subcore_axis_name="s", num_cores=NC, num_subcores=NS
)


@functools.partial(
    pl.kernel,
    out_type=jax.ShapeDtypeStruct((S * D, B), jnp.float32),
    mesh=_mesh,
    scratch_types=[
        pltpu.VMEM((CI,), jnp.int32),
        pltpu.VMEM((CI, D), jnp.float32),
        pltpu.VMEM((D, CI), jnp.float32),
        pltpu.SemaphoreType.DMA,
    ],
    compiler_params=pltpu.CompilerParams(
        use_tc_tiling_on_sc=False, needs_layout_passes=False
    ),
)
def _emb(xt_hbm, tab_hbm, out_hbm, idx_v, rows_g, outb, sem):
    wid = lax.axis_index("s") * NC + lax.axis_index("c")
    iota = lax.iota(jnp.int32, L)
    rows_idx = [iota + g * L for g in range(D // L)]

    def tile(t, carry):
        tid = wid + t * NW
        j = lax.shift_right_logical(tid, 4)
        off = pl.multiple_of(tid * CI, 8)
        pltpu.sync_copy(xt_hbm.at[pl.ds(off, CI)], idx_v)
        pltpu.async_copy(tab_hbm.at[idx_v], rows_g, sem).wait()

        def tok(k, c2):
            col = jnp.broadcast_to(k, (L,)).astype(jnp.int32)
            for g in range(D // L):
                v = rows_g[k, pl.ds(g * L, L)] * SCALE
                plsc.store_scatter(outb, [rows_idx[g], col], v)
            return c2

        lax.fori_loop(0, CI, tok, 0)
        pltpu.sync_copy(
            outb,
            out_hbm.at[pl.ds(j * D, D), pl.ds((tid % 16) * CI, CI)],
        )
        return carry

    lax.fori_loop(0, TPW, tile, 0)


def kernel(x, table):
    xt = x.T.reshape(-1)
    out = _emb(xt, table)
    return out.reshape(S, D, B).transpose(2, 0, 1)


# speedup vs baseline: 1.1527x; 1.1527x over previous
import functools
import jax
import jax.numpy as jnp
from jax import lax
from jax.experimental import pallas as pl
from jax.experimental.pallas import tpu as pltpu
from jax.experimental.pallas import tpu_sc as plsc

NC, NS, L = 2, 16, 16
NW = NC * NS
D = 64
N = 4096 * 200
PER_W = N // NW
C = 256
T = PER_W // C
SCALE = 8.0

_mesh = plsc.VectorSubcoreMesh(
    core_axis_name="c", subcore_axis_name="s", num_cores=NC, num_subcores=NS
)


@functools.partial(
    pl.kernel,
    out_type=jax.ShapeDtypeStruct((N // 2, 2 * D), jnp.float32),
    mesh=_mesh,
    scratch_types=[
        pltpu.VMEM((C,), jnp.int32),
        pltpu.VMEM((C,), jnp.int32),
        pltpu.VMEM((C, 2 * D), jnp.float32),
        pltpu.VMEM((C // 2, 2 * D), jnp.float32),
        pltpu.SemaphoreType.DMA,
    ],
)
def _emb(x_hbm, tab2_hbm, out_hbm, idx_v, pidx_v, rows_g, stage, sem):
    wid = lax.axis_index("s") * NC + lax.axis_index("c")
    base = wid * PER_W

    def chunk(t, carry):
        off = pl.multiple_of(base + t * C, 8)
        pltpu.sync_copy(x_hbm.at[pl.ds(off, C)], idx_v)

        def mk_pidx(g, c2):
            sl = pl.ds(g * L, L)
            pidx_v[sl] = lax.shift_right_logical(idx_v[sl], 1)
            return c2

        lax.fori_loop(0, C // L, mk_pidx, 0)
        pltpu.async_copy(tab2_hbm.at[pidx_v], rows_g, sem).wait()

        def grp(g, c2):
            hv = (idx_v[pl.ds(g * L, L)] & 1) * D
            ssub = stage.at[pl.ds(pl.multiple_of(g * 8, 8), 8)]
            for half in range(2):
                sub = rows_g.at[pl.ds(pl.multiple_of(g * L + half * 8, 8), 8)]
                for l in range(8):
                    h = hv[half * 8 + l]
                    i2l = half * 4 + l // 2
                    p = l % 2
                    for j in range(D // L):
                        dst = pl.ds(p * D + j * L, L)
                        src = pl.ds(h + j * L, L)
                        ssub[i2l, dst] = sub[l, src] * SCALE
            return c2

        lax.fori_loop(0, C // L, grp, 0)
        pltpu.sync_copy(
            stage,
            out_hbm.at[pl.ds(pl.multiple_of((base + t * C) // 2, 8), C // 2)],
        )
        return carry

    lax.fori_loop(0, T, chunk, 0)


def kernel(x, table):
    tab2 = table.reshape(-1, 2 * D)
    out = _emb(x.reshape(-1), tab2)
    return out.reshape(x.shape[0], x.shape[1], D)
